# baseline (speedup 1.0000x reference)
import functools

import jax
import jax.numpy as jnp
from jax.experimental import pallas as pl
from jax.experimental.pallas import tpu as pltpu

_MIN_KEPT = 100000
_DICE_W = 0.5
_BCE_W = 0.5
_SMOOTH = 1.0
_SEARCH_ITERS = 18


def _halve_lanes(v, to=128):
    n = v.shape[-1]
    while n > to:
        n //= 2
        v = v[..., :n] + v[..., n:]
    return v


def _loss_kernel(
    logits_ref, tgt_ref, out_ref, bits_ref, pt_ref, p_ref, t_ref, *, k, B
):
    i = pl.program_id(0)
    l = logits_ref[...]
    t = tgt_ref[...].astype(jnp.float32)
    bce = jnp.maximum(l, 0.0) - l * t + jnp.log1p(jnp.exp(-jnp.abs(l)))
    bits_ref[i] = jax.lax.bitcast_convert_type(bce[0], jnp.int32)
    prob = jax.nn.sigmoid(l)
    pt_ref[i] = jnp.sum(prob * t, axis=(0, 1))
    p_ref[i] = jnp.sum(prob, axis=(0, 1))
    t_ref[i] = jnp.sum(t, axis=(0, 1))

    @pl.when(i == B - 1)
    def _finalize():
        bits = bits_ref[...]
        rows = bits.shape[0] * bits.shape[1]
        cols = bits.shape[2]
        bits2d = bits.reshape(rows, cols)
        ones = jnp.ones((cols, 8), jnp.float32)
        kf = jnp.float32(k)

        def _mxu_count_ge(mid):
            m = (bits2d >= mid).astype(jnp.float32)
            cx = jax.lax.dot_general(
                m, ones, (((1,), (0,)), ((), ())),
                preferred_element_type=jnp.float32,
            )
            return jnp.sum(cx) * 0.125

        def body(_, carry):
            lo, hi = carry
            mid = lo + ((hi - lo + 1) >> 1)
            big = _mxu_count_ge(mid) >= kf
            return jnp.where(big, mid, lo), jnp.where(big, hi, mid - 1)

        lo, _ = jax.lax.fori_loop(
            0, _SEARCH_ITERS, body, (jnp.int32(0), jnp.int32(0x7F7FFFFF))
        )
        tau = jax.lax.bitcast_convert_type(lo, jnp.float32)
        bce_all = jax.lax.bitcast_convert_type(bits, jnp.float32)
        excess = jnp.maximum(bce_all - tau, 0.0)
        bce_sum = jnp.float32(k) * tau + jnp.sum(
            _halve_lanes(jnp.sum(excess, axis=(0, 1)))
        )
        bce_loss = bce_sum / jnp.float32(k)

        spt = jnp.sum(_halve_lanes(pt_ref[...]), axis=-1)
        sp = jnp.sum(_halve_lanes(p_ref[...]), axis=-1)
        st = jnp.sum(_halve_lanes(t_ref[...]), axis=-1)
        num = 2.0 * spt + _SMOOTH
        den = sp + st + _SMOOTH
        dice_loss = jnp.sum(1.0 - num / den) / jnp.float32(B)

        total = _DICE_W * dice_loss + _BCE_W * bce_loss
        out_ref[...] = jnp.broadcast_to(total, (1, 1))


@jax.jit
def kernel(pred_logits, target):
    B = pred_logits.shape[0]
    N = pred_logits.shape[-1] * pred_logits.shape[-2]
    k = min(_MIN_KEPT * B, B * N)
    M = N // 8
    logits = pred_logits.reshape(B, 8, M)
    tgt = target.reshape(B, 8, M)
    out = pl.pallas_call(
        functools.partial(_loss_kernel, k=k, B=B),
        grid=(B,),
        in_specs=[
            pl.BlockSpec((1, 8, M), lambda i: (i, 0, 0)),
            pl.BlockSpec((1, 8, M), lambda i: (i, 0, 0)),
        ],
        out_specs=pl.BlockSpec((1, 1), lambda i: (0, 0)),
        out_shape=jax.ShapeDtypeStruct((1, 1), jnp.float32),
        scratch_shapes=[
            pltpu.VMEM((B, 8, M), jnp.int32),
            pltpu.VMEM((B, M), jnp.float32),
            pltpu.VMEM((B, M), jnp.float32),
            pltpu.VMEM((B, M), jnp.float32),
        ],
    )(logits, tgt)
    return out[0, 0]


# speedup vs baseline: 1.1131x; 1.1131x over previous
import functools

import jax
import jax.numpy as jnp
from jax.experimental import pallas as pl
from jax.experimental.pallas import tpu as pltpu

_MIN_KEPT = 100000
_DICE_W = 0.5
_BCE_W = 0.5
_SMOOTH = 1.0
_SEARCH_ITERS = 16


def _halve_lanes(v, to=128):
    n = v.shape[-1]
    while n > to:
        n //= 2
        v = v[..., :n] + v[..., n:]
    return v


def _loss_kernel(
    logits_ref, tgt_ref, out_ref, bits_ref, pt_ref, p_ref, t_ref, *, k, B
):
    i = pl.program_id(0)
    l = logits_ref[...]
    t = tgt_ref[...].astype(jnp.float32)
    bce = jnp.maximum(l, 0.0) - l * t + jnp.log1p(jnp.exp(-jnp.abs(l)))
    bits_ref[i] = jax.lax.bitcast_convert_type(bce[0], jnp.int32)
    prob = jax.nn.sigmoid(l)
    pt_ref[i] = jnp.sum(prob * t, axis=(0, 1))
    p_ref[i] = jnp.sum(prob, axis=(0, 1))
    t_ref[i] = jnp.sum(t, axis=(0, 1))

    @pl.when(i == B - 1)
    def _finalize():
        bits = bits_ref[...]

        def body(_, carry):
            lo, hi = carry
            mid = lo + ((hi - lo + 1) >> 1)
            m = (bits >= mid).astype(jnp.int32)
            c = jnp.sum(_halve_lanes(jnp.sum(m, axis=(0, 1))))
            big = c >= k
            return jnp.where(big, mid, lo), jnp.where(big, hi, mid - 1)

        lo, _ = jax.lax.fori_loop(
            0, _SEARCH_ITERS, body, (jnp.int32(0), jnp.int32(0x7F7FFFFF))
        )
        tau = jax.lax.bitcast_convert_type(lo, jnp.float32)
        bce_all = jax.lax.bitcast_convert_type(bits, jnp.float32)
        excess = jnp.maximum(bce_all - tau, 0.0)
        bce_sum = jnp.float32(k) * tau + jnp.sum(
            _halve_lanes(jnp.sum(excess, axis=(0, 1)))
        )
        bce_loss = bce_sum / jnp.float32(k)

        spt = jnp.sum(_halve_lanes(pt_ref[...]), axis=-1)
        sp = jnp.sum(_halve_lanes(p_ref[...]), axis=-1)
        st = jnp.sum(_halve_lanes(t_ref[...]), axis=-1)
        num = 2.0 * spt + _SMOOTH
        den = sp + st + _SMOOTH
        dice_loss = jnp.sum(1.0 - num / den) / jnp.float32(B)

        total = _DICE_W * dice_loss + _BCE_W * bce_loss
        out_ref[...] = jnp.broadcast_to(total, (1, 1))


@jax.jit
def kernel(pred_logits, target):
    B = pred_logits.shape[0]
    N = pred_logits.shape[-1] * pred_logits.shape[-2]
    k = min(_MIN_KEPT * B, B * N)
    M = N // 8
    logits = pred_logits.reshape(B, 8, M)
    tgt = target.astype(jnp.bfloat16).reshape(B, 8, M)
    out = pl.pallas_call(
        functools.partial(_loss_kernel, k=k, B=B),
        grid=(B,),
        in_specs=[
            pl.BlockSpec((1, 8, M), lambda i: (i, 0, 0)),
            pl.BlockSpec((1, 8, M), lambda i: (i, 0, 0)),
        ],
        out_specs=pl.BlockSpec((1, 1), lambda i: (0, 0)),
        out_shape=jax.ShapeDtypeStruct((1, 1), jnp.float32),
        scratch_shapes=[
            pltpu.VMEM((B, 8, M), jnp.int32),
            pltpu.VMEM((B, M), jnp.float32),
            pltpu.VMEM((B, M), jnp.float32),
            pltpu.VMEM((B, M), jnp.float32),
        ],
    )(logits, tgt)
    return out[0, 0]
